# baseline (speedup 1.0000x reference)
import functools

import jax
import jax.numpy as jnp
from jax import lax
from jax.experimental import pallas as pl
from jax.experimental.pallas import tpu as pltpu
from jax.experimental.pallas import tpu_sc as plsc

B = 16384
K = 5
D = 64
CB = 128
NC, NS = 2, 16
NW = NC * NS
BPW = B // NW
NCH = BPW // CB
NG = CB // 16

_mesh = plsc.VectorSubcoreMesh(core_axis_name="c", subcore_axis_name="s")


@functools.partial(
    pl.kernel,
    out_type=jax.ShapeDtypeStruct((K, B // 16, 16), jnp.float32),
    mesh=_mesh,
    scratch_types=[
        pltpu.VMEM((2, 1 + K, CB), jnp.int32),
        pltpu.VMEM((2, CB, D), jnp.float32),
        pltpu.VMEM((2, K, CB, D), jnp.float32),
        pltpu.VMEM((K, BPW // 16, 16), jnp.float32),
        pltpu.SemaphoreType.DMA,
        pltpu.SemaphoreType.DMA,
    ],
    compiler_params=pltpu.CompilerParams(needs_layout_passes=False,
                                         use_tc_tiling_on_sc=False),
)
def _sc_skipgram(idx_hbm, ttab_hbm, otab_hbm, out_hbm,
                 idx_v, trows_v, crows_v, out_v, sem0, sem1):
    wid = lax.axis_index("s") * NC + lax.axis_index("c")
    g0 = wid * NCH
    sems = [sem0, sem1]
    descs = [None, None]

    def fire(c):
        s = c % 2
        pltpu.sync_copy(idx_hbm.at[g0 + c], idx_v.at[s])
        ds = [pltpu.async_copy(ttab_hbm.at[idx_v.at[s, 0]], trows_v.at[s],
                               sems[s])]
        for k in range(K):
            ds.append(pltpu.async_copy(otab_hbm.at[idx_v.at[s, 1 + k]],
                                       crows_v.at[s, k], sems[s]))
        descs[s] = ds

    def compute(c):
        s = c % 2
        trows = trows_v.at[s]
        crows = crows_v.at[s]
        for g in range(NG):
            rowg = lax.iota(jnp.int32, 16) + g * 16

            def body(d, accs):
                dvec = lax.broadcast(d, (16,))
                tcol = plsc.load_gather(trows, [rowg, dvec])
                return tuple(
                    accs[k] + tcol * plsc.load_gather(crows.at[k], [rowg, dvec])
                    for k in range(K))

            accs = lax.fori_loop(
                0, D, body,
                tuple(jnp.zeros((16,), jnp.float32) for _ in range(K)))
            for k in range(K):
                out_v[k, c * NG + g] = accs[k]

    fire(0)
    for c in range(NCH):
        if c + 1 < NCH:
            fire(c + 1)
        for d in descs[c % 2]:
            d.wait()
        compute(c)
    for k in range(K):
        pltpu.sync_copy(out_v.at[k],
                        out_hbm.at[k, pl.ds(wid * (BPW // 16), BPW // 16)])


def kernel(target, context, target_table, output_table):
    tgt = target.astype(jnp.int32).reshape(B // CB, 1, CB)
    ctx = (context.astype(jnp.int32).T
           .reshape(K, B // CB, CB).transpose(1, 0, 2))
    idx_all = jnp.concatenate([tgt, ctx], axis=1)
    out3 = _sc_skipgram(idx_all, target_table, output_table)
    return out3.reshape(K, B).T
